# baseline (speedup 1.0000x reference)
import jax
import jax.numpy as jnp
from jax.experimental import pallas as pl
from jax.experimental.pallas import tpu as pltpu

_B = 16384
_C = 1000
_CP = 1024
_SMOOTH = 0.1
_CONF = 0.9
_ROWS = 2048
_GRID = _B // _ROWS


def _half_sum(pred, tgt, aug_ref):
    m = jnp.max(pred, axis=1, keepdims=True)
    lse = m + jnp.log(jnp.sum(jnp.exp(pred - m), axis=1, keepdims=True))

    cols = jax.lax.broadcasted_iota(jnp.int32, (_ROWS, _CP), 1)
    onehot = (cols == tgt[:, None])
    g = jnp.dot(onehot.astype(jnp.bfloat16), aug_ref[...],
                preferred_element_type=jnp.float32)

    gr = g[:, _C:_C + 1]
    gd = g[:, _C + 1:_C + 2]
    c = _CONF - gd
    t_row = gr + c

    td = g[:, 0:_C] + jnp.where(onehot[:, 0:_C], c, 0.0)
    return jnp.sum(lse * t_row) - jnp.sum(td * pred)


def _loss_kernel(tgta_ref, tgtb_ref, preda_ref, predb_ref, sim_ref,
                 out_ref, aug_ref):
    i = pl.program_id(0)

    @pl.when(i == 0)
    def _build_aug():
        sim = sim_ref[...]
        aug_ref[...] = jnp.zeros((_CP, _CP), jnp.bfloat16)
        aug_ref[0:_C, 0:_C] = sim
        rs = jnp.sum(sim.astype(jnp.float32), axis=1, keepdims=True)
        eye = (jax.lax.broadcasted_iota(jnp.int32, (_C, _C), 0)
               == jax.lax.broadcasted_iota(jnp.int32, (_C, _C), 1))
        dg = jnp.sum(jnp.where(eye, sim, jnp.bfloat16(0)).astype(jnp.float32),
                     axis=1, keepdims=True)
        aug_ref[0:_C, _C:_C + 1] = rs.astype(jnp.bfloat16)
        aug_ref[0:_C, _C + 1:_C + 2] = dg.astype(jnp.bfloat16)

    sa = _half_sum(preda_ref[0], tgta_ref[0, 0, 0, :], aug_ref)
    sb = _half_sum(predb_ref[0], tgtb_ref[0, 0, 0, :], aug_ref)
    block_sum = (sa + sb) * (1.0 / _B)

    @pl.when(i == 0)
    def _init():
        out_ref[...] = jnp.zeros((1, 1), jnp.float32)

    out_ref[...] += jnp.full((1, 1), block_sum, jnp.float32)


@jax.jit
def kernel(pred, target, similarity):
    half_grid = _GRID // 2
    tgt4 = target.reshape(2, half_grid, 1, _ROWS)
    pred3 = pred.reshape(2, _B // 2, _C)
    sim_bf = (similarity * _SMOOTH).astype(jnp.bfloat16)
    out = pl.pallas_call(
        _loss_kernel,
        grid=(half_grid,),
        in_specs=[
            pl.BlockSpec((1, 1, 1, _ROWS), lambda i: (0, i, 0, 0)),
            pl.BlockSpec((1, 1, 1, _ROWS), lambda i: (1, i, 0, 0)),
            pl.BlockSpec((1, _ROWS, _C), lambda i: (0, i, 0)),
            pl.BlockSpec((1, _ROWS, _C), lambda i: (1, i, 0)),
            pl.BlockSpec(memory_space=pltpu.VMEM),
        ],
        out_specs=pl.BlockSpec((1, 1), lambda i: (0, 0)),
        out_shape=jax.ShapeDtypeStruct((1, 1), jnp.float32),
        scratch_shapes=[pltpu.VMEM((_CP, _CP), jnp.bfloat16)],
        compiler_params=pltpu.CompilerParams(
            dimension_semantics=("arbitrary",),
        ),
    )(tgt4, tgt4, pred3, pred3, sim_bf)
    return out[0, 0]


# speedup vs baseline: 1.0300x; 1.0300x over previous
import jax
import jax.numpy as jnp
from jax.experimental import pallas as pl
from jax.experimental.pallas import tpu as pltpu

_B = 16384
_C = 1000
_SMOOTH = 0.1
_CONF = 0.9
_ROWS = 2048
_GRID = _B // _ROWS


def _loss_kernel(tgt_ref, pred_ref, sim_ref, out_ref):
    i = pl.program_id(0)
    pred = pred_ref[...]
    tgt = tgt_ref[0, 0, :]

    m = jnp.max(pred, axis=1, keepdims=True)
    lse = m + jnp.log(jnp.sum(jnp.exp(pred - m), axis=1, keepdims=True))

    cols = jax.lax.broadcasted_iota(jnp.int32, (_ROWS, _C), 1)
    onehot = (cols == tgt[:, None])
    gathered = jnp.dot(onehot.astype(jnp.bfloat16), sim_ref[...],
                       preferred_element_type=jnp.float32)

    td = jnp.where(onehot, _CONF, gathered)
    t_row = jnp.sum(td, axis=1, keepdims=True)
    u_all = jnp.sum(td * pred)
    block_sum = (jnp.sum(lse * t_row) - u_all) * (1.0 / _B)

    @pl.when(i == 0)
    def _init():
        out_ref[...] = jnp.zeros((1, 1), jnp.float32)

    out_ref[...] += jnp.full((1, 1), block_sum, jnp.float32)


@jax.jit
def kernel(pred, target, similarity):
    tgt3 = target.reshape(_GRID, 1, _ROWS)
    sim_bf = (similarity * _SMOOTH).astype(jnp.bfloat16)
    out = pl.pallas_call(
        _loss_kernel,
        grid=(_GRID,),
        in_specs=[
            pl.BlockSpec((1, 1, _ROWS), lambda i: (i, 0, 0)),
            pl.BlockSpec((_ROWS, _C), lambda i: (i, 0)),
            pl.BlockSpec(memory_space=pltpu.VMEM),
        ],
        out_specs=pl.BlockSpec((1, 1), lambda i: (0, 0)),
        out_shape=jax.ShapeDtypeStruct((1, 1), jnp.float32),
        compiler_params=pltpu.CompilerParams(
            dimension_semantics=("arbitrary",),
        ),
    )(tgt3, pred, sim_bf)
    return out[0, 0]
